# baseline (speedup 1.0000x reference)
import functools

import jax
import jax.numpy as jnp
from jax import lax
from jax.experimental import pallas as pl
from jax.experimental.pallas import tpu as pltpu
from jax.experimental.pallas import tpu_sc as plsc

_NFRAMES = 4
_NALL = 1024
_NLOC = 512
_NNEI = 64
_NTYPES = 4
_NSPLINE = 1024
_RCUT = 6.0

_NC = 2
_NS = 16
_L = 16
_NW = _NC * _NS
_ATOMS_W = _NFRAMES * _NLOC // _NW
_GROUPS = _ATOMS_W // _L
_W_PER_FRAME = _NLOC // _ATOMS_W
_NEI_W = _ATOMS_W * _NNEI
_NPAIR = _NFRAMES * _NLOC * _NNEI
_NROWS = _NTYPES * _NTYPES * _NSPLINE


def _sc_phase1(coordx_hbm, atype_hbm, nlist_hbm, d2_hbm, tb_hbm,
               xs_v, ys_v, zs_v, atype_v, nlist_v, d2_v, tb_v, sem):
    wid = lax.axis_index("s") * _NC + lax.axis_index("c")
    frame = wid // _W_PER_FRAME

    copies = [
        pltpu.make_async_copy(nlist_hbm.at[pl.ds(wid * _NEI_W, _NEI_W)],
                              nlist_v, sem),
        pltpu.make_async_copy(coordx_hbm.at[pl.ds(frame * _NALL, _NALL)],
                              xs_v, sem),
        pltpu.make_async_copy(
            coordx_hbm.at[pl.ds((_NFRAMES + frame) * _NALL, _NALL)], ys_v, sem),
        pltpu.make_async_copy(
            coordx_hbm.at[pl.ds((2 * _NFRAMES + frame) * _NALL, _NALL)],
            zs_v, sem),
        pltpu.make_async_copy(atype_hbm.at[pl.ds(frame * _NALL, _NALL)],
                              atype_v, sem),
    ]
    for c in copies:
        c.start()
    for c in copies:
        c.wait()

    lane = lax.iota(jnp.int32, _L)
    for g in range(_GROUPS):
        loc = (wid % _W_PER_FRAME) * _ATOMS_W + g * _L + lane
        xi = plsc.load_gather(xs_v, [loc])
        yi = plsc.load_gather(ys_v, [loc])
        zi = plsc.load_gather(zs_v, [loc])
        itype = plsc.load_gather(atype_v, [loc])
        itab = itype * (_NTYPES * _NSPLINE)
        nbase = (g * _L + lane) * _NNEI

        @plsc.parallel_loop(0, _NNEI, unroll=4)
        def body(k):
            jn = plsc.load_gather(nlist_v, [nbase + k])
            j = jnp.maximum(jn, 0)
            xj = plsc.load_gather(xs_v, [j])
            yj = plsc.load_gather(ys_v, [j])
            zj = plsc.load_gather(zs_v, [j])
            jtype = plsc.load_gather(atype_v, [j])
            dx = xi - xj
            dy = yi - yj
            dz = zi - zj
            d2 = dx * dx + dy * dy + dz * dz
            tb = itab + jtype * _NSPLINE
            tb = jnp.where(jn == -1, jnp.full((_L,), -1, jnp.int32), tb)
            slot = k * _ATOMS_W + g * _L
            d2_v[pl.ds(slot, _L)] = d2
            tb_v[pl.ds(slot, _L)] = tb

    pltpu.sync_copy(d2_v, d2_hbm.at[pl.ds(wid * _NEI_W, _NEI_W)])
    pltpu.sync_copy(tb_v, tb_hbm.at[pl.ds(wid * _NEI_W, _NEI_W)])


def _tc_phase2(d2_ref, tb_ref, info_ref, frac_ref, gidx_ref):
    rmin = info_ref[0, 0]
    hi = info_ref[0, 1]
    d2 = d2_ref[...]
    tb = tb_ref[...]
    rr = jnp.sqrt(d2)
    uu = (rr - rmin) * hi
    uu = jnp.where(tb < 0, jnp.float32(_NSPLINE + 1), uu)
    idx = uu.astype(jnp.int32)
    frac = uu - idx.astype(jnp.float32)
    cidx = jnp.clip(idx, 0, _NSPLINE - 1)
    valid = (idx <= _NSPLINE) & (rr <= _RCUT) & (tb >= 0)
    gidx = jnp.where(valid, tb + cidx, jnp.int32(-1))
    frac_ref[...] = frac
    gidx_ref[...] = gidx


def _sc_phase3(frac_hbm, gidx_hbm, tab_hbm, out_hbm,
               frac_v, gidx_v, a3_v, a2_v, a1_v, a0_v, out_v, sem):
    wid = lax.axis_index("s") * _NC + lax.axis_index("c")

    copies = [
        pltpu.make_async_copy(frac_hbm.at[pl.ds(wid * _NEI_W, _NEI_W)],
                              frac_v, sem),
        pltpu.make_async_copy(gidx_hbm.at[pl.ds(wid * _NEI_W, _NEI_W)],
                              gidx_v, sem),
        pltpu.make_async_copy(tab_hbm.at[pl.ds(0, _NROWS)], a3_v, sem),
        pltpu.make_async_copy(tab_hbm.at[pl.ds(_NROWS, _NROWS)], a2_v, sem),
        pltpu.make_async_copy(tab_hbm.at[pl.ds(2 * _NROWS, _NROWS)], a1_v, sem),
        pltpu.make_async_copy(tab_hbm.at[pl.ds(3 * _NROWS, _NROWS)], a0_v, sem),
    ]
    for c in copies:
        c.start()
    for c in copies:
        c.wait()

    zero = jnp.zeros((_L,), jnp.float32)
    for g in range(_GROUPS):

        @plsc.parallel_loop(0, _NNEI, unroll=4,
                            carry=jnp.zeros((_L,), jnp.float32))
        def acc(k, acc):
            slot = k * _ATOMS_W + g * _L
            frac = frac_v[pl.ds(slot, _L)]
            gidx = gidx_v[pl.ds(slot, _L)]
            gi = jnp.maximum(gidx, 0)
            a3 = plsc.load_gather(a3_v, [gi])
            a2 = plsc.load_gather(a2_v, [gi])
            a1 = plsc.load_gather(a1_v, [gi])
            a0 = plsc.load_gather(a0_v, [gi])
            e = ((a3 * frac + a2) * frac + a1) * frac + a0
            e = jnp.where(gidx < 0, zero, e)
            return acc + e

        out_v[pl.ds(g * _L, _L)] = 0.5 * acc

    pltpu.sync_copy(out_v, out_hbm.at[pl.ds(wid * _ATOMS_W, _ATOMS_W)])


@jax.jit
def _pair_tab(coord_pl, atype_flat, nlist_flat, tab_pl, info11):
    mesh = plsc.VectorSubcoreMesh(core_axis_name="c", subcore_axis_name="s")
    sc_params = pltpu.CompilerParams(needs_layout_passes=False)

    d2_flat, tb_flat = functools.partial(
        pl.kernel,
        mesh=mesh,
        out_type=(jax.ShapeDtypeStruct((_NPAIR,), jnp.float32),
                  jax.ShapeDtypeStruct((_NPAIR,), jnp.int32)),
        scratch_types=[
            pltpu.VMEM((_NALL,), jnp.float32),
            pltpu.VMEM((_NALL,), jnp.float32),
            pltpu.VMEM((_NALL,), jnp.float32),
            pltpu.VMEM((_NALL,), jnp.int32),
            pltpu.VMEM((_NEI_W,), jnp.int32),
            pltpu.VMEM((_NEI_W,), jnp.float32),
            pltpu.VMEM((_NEI_W,), jnp.int32),
            pltpu.SemaphoreType.DMA,
        ],
        compiler_params=sc_params,
    )(_sc_phase1)(coord_pl, atype_flat, nlist_flat)

    frac_flat, gidx_flat = pl.pallas_call(
        _tc_phase2,
        out_shape=(jax.ShapeDtypeStruct((_NPAIR // 128, 128), jnp.float32),
                   jax.ShapeDtypeStruct((_NPAIR // 128, 128), jnp.int32)),
    )(d2_flat.reshape(_NPAIR // 128, 128),
      tb_flat.reshape(_NPAIR // 128, 128), info11)

    out = functools.partial(
        pl.kernel,
        mesh=mesh,
        out_type=jax.ShapeDtypeStruct((_NFRAMES * _NLOC,), jnp.float32),
        scratch_types=[
            pltpu.VMEM((_NEI_W,), jnp.float32),
            pltpu.VMEM((_NEI_W,), jnp.int32),
            pltpu.VMEM((_NROWS,), jnp.float32),
            pltpu.VMEM((_NROWS,), jnp.float32),
            pltpu.VMEM((_NROWS,), jnp.float32),
            pltpu.VMEM((_NROWS,), jnp.float32),
            pltpu.VMEM((_ATOMS_W,), jnp.float32),
            pltpu.SemaphoreType.DMA,
        ],
        compiler_params=sc_params,
    )(_sc_phase3)(frac_flat.reshape(_NPAIR), gidx_flat.reshape(_NPAIR),
                  tab_pl)
    return out


def kernel(extended_coord, extended_atype, nlist, tab_data, tab_info):
    nframes, nloc, nnei = nlist.shape
    coord_pl = jnp.transpose(extended_coord, (2, 0, 1)).reshape(-1)
    atype_flat = extended_atype.reshape(-1)
    nlist_flat = nlist.reshape(-1)
    tab_pl = jnp.moveaxis(tab_data, 3, 0).reshape(-1)
    info11 = jnp.stack([tab_info[0], 1.0 / tab_info[1]]).reshape(1, 2)
    out = _pair_tab(coord_pl, atype_flat, nlist_flat, tab_pl, info11)
    return out.reshape(nframes, nloc)


# speedup vs baseline: 1.0050x; 1.0050x over previous
import functools

import jax
import jax.numpy as jnp
from jax import lax
from jax.experimental import pallas as pl
from jax.experimental.pallas import tpu as pltpu
from jax.experimental.pallas import tpu_sc as plsc

_NFRAMES = 4
_NALL = 1024
_NLOC = 512
_NNEI = 64
_NTYPES = 4
_NSPLINE = 1024
_RCUT = 6.0

_NC = 2
_NS = 16
_L = 16
_NW = _NC * _NS
_ATOMS_W = _NFRAMES * _NLOC // _NW
_GROUPS = _ATOMS_W // _L
_W_PER_FRAME = _NLOC // _ATOMS_W
_NEI_W = _ATOMS_W * _NNEI
_NPAIR = _NFRAMES * _NLOC * _NNEI
_NROWS = _NTYPES * _NTYPES * _NSPLINE


def _sc_phase1(coordx_hbm, atype_hbm, nlist_hbm, d2_hbm, tb_hbm,
               xs_v, ys_v, zs_v, atype_v, nlist_v, d2_v, tb_v, sem):
    wid = lax.axis_index("s") * _NC + lax.axis_index("c")
    frame = wid // _W_PER_FRAME

    copies = [
        pltpu.make_async_copy(nlist_hbm.at[pl.ds(wid * _NEI_W, _NEI_W)],
                              nlist_v, sem),
        pltpu.make_async_copy(coordx_hbm.at[pl.ds(frame * _NALL, _NALL)],
                              xs_v, sem),
        pltpu.make_async_copy(
            coordx_hbm.at[pl.ds((_NFRAMES + frame) * _NALL, _NALL)], ys_v, sem),
        pltpu.make_async_copy(
            coordx_hbm.at[pl.ds((2 * _NFRAMES + frame) * _NALL, _NALL)],
            zs_v, sem),
        pltpu.make_async_copy(atype_hbm.at[pl.ds(frame * _NALL, _NALL)],
                              atype_v, sem),
    ]
    for c in copies:
        c.start()
    for c in copies:
        c.wait()

    lane = lax.iota(jnp.int32, _L)
    for g in range(_GROUPS):
        loc = (wid % _W_PER_FRAME) * _ATOMS_W + g * _L + lane
        xi = plsc.load_gather(xs_v, [loc])
        yi = plsc.load_gather(ys_v, [loc])
        zi = plsc.load_gather(zs_v, [loc])
        itype = plsc.load_gather(atype_v, [loc])
        itab = itype * (_NTYPES * _NSPLINE)
        nbase = (g * _L + lane) * _NNEI

        @plsc.parallel_loop(0, _NNEI, unroll=4)
        def body(k):
            jn = plsc.load_gather(nlist_v, [nbase + k])
            j = jnp.maximum(jn, 0)
            xj = plsc.load_gather(xs_v, [j])
            yj = plsc.load_gather(ys_v, [j])
            zj = plsc.load_gather(zs_v, [j])
            jtype = plsc.load_gather(atype_v, [j])
            dx = xi - xj
            dy = yi - yj
            dz = zi - zj
            d2 = dx * dx + dy * dy + dz * dz
            tb = itab + jtype * _NSPLINE
            tb = jnp.where(jn == -1, jnp.full((_L,), -1, jnp.int32), tb)
            slot = k * _ATOMS_W + g * _L
            d2_v[pl.ds(slot, _L)] = d2
            tb_v[pl.ds(slot, _L)] = tb

    out_copies = [
        pltpu.make_async_copy(d2_v, d2_hbm.at[pl.ds(wid * _NEI_W, _NEI_W)],
                              sem),
        pltpu.make_async_copy(tb_v, tb_hbm.at[pl.ds(wid * _NEI_W, _NEI_W)],
                              sem),
    ]
    for c in out_copies:
        c.start()
    for c in out_copies:
        c.wait()


def _tc_phase2(d2_ref, tb_ref, info_ref, frac_ref, gidx_ref):
    rmin = info_ref[0, 0]
    hi = info_ref[0, 1]
    d2 = d2_ref[...]
    tb = tb_ref[...]
    rr = jnp.sqrt(d2)
    uu = (rr - rmin) * hi
    uu = jnp.where(tb < 0, jnp.float32(_NSPLINE + 1), uu)
    idx = uu.astype(jnp.int32)
    frac = uu - idx.astype(jnp.float32)
    cidx = jnp.clip(idx, 0, _NSPLINE - 1)
    valid = (idx <= _NSPLINE) & (rr <= _RCUT) & (tb >= 0)
    gidx = jnp.where(valid, tb + cidx, jnp.int32(-1))
    frac_ref[...] = frac
    gidx_ref[...] = gidx


def _sc_phase3(frac_hbm, gidx_hbm, tab_hbm, out_hbm,
               frac_v, gidx_v, a3_v, a2_v, a1_v, a0_v, out_v, sem):
    wid = lax.axis_index("s") * _NC + lax.axis_index("c")

    copies = [
        pltpu.make_async_copy(frac_hbm.at[pl.ds(wid * _NEI_W, _NEI_W)],
                              frac_v, sem),
        pltpu.make_async_copy(gidx_hbm.at[pl.ds(wid * _NEI_W, _NEI_W)],
                              gidx_v, sem),
        pltpu.make_async_copy(tab_hbm.at[pl.ds(0, _NROWS)], a3_v, sem),
        pltpu.make_async_copy(tab_hbm.at[pl.ds(_NROWS, _NROWS)], a2_v, sem),
        pltpu.make_async_copy(tab_hbm.at[pl.ds(2 * _NROWS, _NROWS)], a1_v, sem),
        pltpu.make_async_copy(tab_hbm.at[pl.ds(3 * _NROWS, _NROWS)], a0_v, sem),
    ]
    for c in copies:
        c.start()
    for c in copies:
        c.wait()

    zero = jnp.zeros((_L,), jnp.float32)
    for g in range(_GROUPS):

        @plsc.parallel_loop(0, _NNEI, unroll=4,
                            carry=jnp.zeros((_L,), jnp.float32))
        def acc(k, acc):
            slot = k * _ATOMS_W + g * _L
            frac = frac_v[pl.ds(slot, _L)]
            gidx = gidx_v[pl.ds(slot, _L)]
            gi = jnp.maximum(gidx, 0)
            a3 = plsc.load_gather(a3_v, [gi])
            a2 = plsc.load_gather(a2_v, [gi])
            a1 = plsc.load_gather(a1_v, [gi])
            a0 = plsc.load_gather(a0_v, [gi])
            e = ((a3 * frac + a2) * frac + a1) * frac + a0
            e = jnp.where(gidx < 0, zero, e)
            return acc + e

        out_v[pl.ds(g * _L, _L)] = 0.5 * acc

    pltpu.sync_copy(out_v, out_hbm.at[pl.ds(wid * _ATOMS_W, _ATOMS_W)])


@jax.jit
def _pair_tab(coord_pl, atype_flat, nlist_flat, tab_pl, info11):
    mesh = plsc.VectorSubcoreMesh(core_axis_name="c", subcore_axis_name="s")
    sc_params = pltpu.CompilerParams(needs_layout_passes=False)

    d2_flat, tb_flat = functools.partial(
        pl.kernel,
        mesh=mesh,
        out_type=(jax.ShapeDtypeStruct((_NPAIR,), jnp.float32),
                  jax.ShapeDtypeStruct((_NPAIR,), jnp.int32)),
        scratch_types=[
            pltpu.VMEM((_NALL,), jnp.float32),
            pltpu.VMEM((_NALL,), jnp.float32),
            pltpu.VMEM((_NALL,), jnp.float32),
            pltpu.VMEM((_NALL,), jnp.int32),
            pltpu.VMEM((_NEI_W,), jnp.int32),
            pltpu.VMEM((_NEI_W,), jnp.float32),
            pltpu.VMEM((_NEI_W,), jnp.int32),
            pltpu.SemaphoreType.DMA,
        ],
        compiler_params=sc_params,
    )(_sc_phase1)(coord_pl, atype_flat, nlist_flat)

    frac_flat, gidx_flat = pl.pallas_call(
        _tc_phase2,
        out_shape=(jax.ShapeDtypeStruct((_NPAIR // 128, 128), jnp.float32),
                   jax.ShapeDtypeStruct((_NPAIR // 128, 128), jnp.int32)),
    )(d2_flat.reshape(_NPAIR // 128, 128),
      tb_flat.reshape(_NPAIR // 128, 128), info11)

    out = functools.partial(
        pl.kernel,
        mesh=mesh,
        out_type=jax.ShapeDtypeStruct((_NFRAMES * _NLOC,), jnp.float32),
        scratch_types=[
            pltpu.VMEM((_NEI_W,), jnp.float32),
            pltpu.VMEM((_NEI_W,), jnp.int32),
            pltpu.VMEM((_NROWS,), jnp.float32),
            pltpu.VMEM((_NROWS,), jnp.float32),
            pltpu.VMEM((_NROWS,), jnp.float32),
            pltpu.VMEM((_NROWS,), jnp.float32),
            pltpu.VMEM((_ATOMS_W,), jnp.float32),
            pltpu.SemaphoreType.DMA,
        ],
        compiler_params=sc_params,
    )(_sc_phase3)(frac_flat.reshape(_NPAIR), gidx_flat.reshape(_NPAIR),
                  tab_pl)
    return out


def kernel(extended_coord, extended_atype, nlist, tab_data, tab_info):
    nframes, nloc, nnei = nlist.shape
    coord_pl = jnp.transpose(extended_coord, (2, 0, 1)).reshape(-1)
    atype_flat = extended_atype.reshape(-1)
    nlist_flat = nlist.reshape(-1)
    tab_pl = jnp.moveaxis(tab_data, 3, 0).reshape(-1)
    info11 = jnp.stack([tab_info[0], 1.0 / tab_info[1]]).reshape(1, 2)
    out = _pair_tab(coord_pl, atype_flat, nlist_flat, tab_pl, info11)
    return out.reshape(nframes, nloc)
